# baseline (speedup 1.0000x reference)
import jax
import jax.numpy as jnp
from jax import lax
from jax.experimental import pallas as pl
from jax.experimental.pallas import tpu as pltpu
from jax.experimental.pallas import tpu_sc as plsc

BATCH = 16384
EMBED = 64
NUM_CORES = 2
NUM_SUBCORES = 16
NUM_TILES = NUM_CORES * NUM_SUBCORES
B_PER_TILE = BATCH // NUM_TILES


def kernel(inputs, W):
    idx = inputs.reshape((BATCH,))

    mesh = plsc.VectorSubcoreMesh(core_axis_name="c", subcore_axis_name="s")

    @pl.kernel(
        out_type=jax.ShapeDtypeStruct((BATCH, EMBED), W.dtype),
        mesh=mesh,
        scratch_types=[
            pltpu.VMEM((B_PER_TILE,), jnp.int32),
            pltpu.VMEM((B_PER_TILE, EMBED), jnp.float32),
            pltpu.SemaphoreType.DMA,
        ],
        compiler_params=pltpu.CompilerParams(use_tc_tiling_on_sc=False),
    )
    def gather_kernel(table_hbm, idx_hbm, out_hbm, idx_v, rows_v, sem):
        wid = lax.axis_index("s") * NUM_CORES + lax.axis_index("c")
        base = wid * B_PER_TILE
        pltpu.sync_copy(idx_hbm.at[pl.ds(base, B_PER_TILE)], idx_v)
        pltpu.async_copy(table_hbm.at[idx_v], rows_v, sem).wait()
        pltpu.sync_copy(rows_v, out_hbm.at[pl.ds(base, B_PER_TILE)])

    return gather_kernel(W, idx)


# speedup vs baseline: 1.0307x; 1.0307x over previous
import jax
import jax.numpy as jnp
from jax import lax
from jax.experimental import pallas as pl
from jax.experimental.pallas import tpu as pltpu
from jax.experimental.pallas import tpu_sc as plsc

BATCH = 16384
EMBED = 64
NUM_CORES = 2
NUM_SUBCORES = 16
NUM_TILES = NUM_CORES * NUM_SUBCORES
B_PER_TILE = BATCH // NUM_TILES
LANES = 16
N_CHUNKS = B_PER_TILE // LANES


def kernel(inputs, W):
    idx = inputs.reshape((BATCH,))

    mesh = plsc.VectorSubcoreMesh(core_axis_name="c", subcore_axis_name="s")

    @pl.kernel(
        out_type=jax.ShapeDtypeStruct((BATCH, EMBED), W.dtype),
        mesh=mesh,
        scratch_types=[
            pltpu.VMEM((B_PER_TILE,), jnp.int32),
            pltpu.SemaphoreType.DMA,
            pltpu.SemaphoreType.DMA,
        ],
        compiler_params=pltpu.CompilerParams(needs_layout_passes=False),
    )
    def gather_kernel(table_hbm, idx_hbm, out_hbm, idx_v, sem_i, sem):
        wid = lax.axis_index("s") * NUM_CORES + lax.axis_index("c")
        base = wid * B_PER_TILE
        pltpu.async_copy(idx_hbm.at[pl.ds(base, B_PER_TILE)], idx_v, sem_i).wait()

        lane = lax.broadcasted_iota(jnp.int32, (LANES,), 0)

        @pl.loop(0, N_CHUNKS)
        def _(c):
            chunk = idx_v[pl.ds(c * LANES, LANES)]
            for j in range(LANES):
                i = jnp.sum(jnp.where(lane == j, chunk, 0))
                pltpu.make_async_copy(
                    table_hbm.at[pl.ds(i, 1)],
                    out_hbm.at[pl.ds(base + c * LANES + j, 1)],
                    sem,
                ).start()

        @pl.loop(0, B_PER_TILE)
        def _(b):
            pltpu.make_async_copy(
                table_hbm.at[pl.ds(0, 1)],
                out_hbm.at[pl.ds(base + b, 1)],
                sem,
            ).wait()

    return gather_kernel(W, idx)
